# baseline (speedup 1.0000x reference)
import functools

import numpy as np
import jax
import jax.numpy as jnp
from jax import lax
from jax.experimental import pallas as pl
from jax.experimental.pallas import tpu as pltpu
from jax.experimental.pallas import tpu_sc as plsc

N = 10000
C = 128
E = 320000
HALF = N // 2
GROUPS = 8
GSIZE = C // GROUPS
EPS = 1e-5

NCORES = 2
NSUB = 16
CHUNK = 80
GCH = 4
NG = 64
NCH = NG * GCH
EPT = NCH * CHUNK
E_PAD = EPT * NSUB
NPAIR = NG // 2


def _edge_body(hp_hbm, src_hbm, dst_hbm, ew_hbm, pb_hbm, out_hbm,
               si0, si1, di0, di1, wi0, wi1, pb0, pb1,
               gb0, gb1, ms0, ms1, table, hp_sh, wsm, bsm,
               isem0, isem1, gsem0, gsem1, ssem0, ssem1):
    c = lax.axis_index("c")
    s = lax.axis_index("s")
    si = [si0, si1]
    di = [di0, di1]
    wi = [wi0, wi1]
    pb = [pb0, pb1]
    gb = [gb0, gb1]
    ms = [ms0, ms1]
    isem = [isem0, isem1]
    gsem = [gsem0, gsem1]
    ssem = [ssem0, ssem1]

    @pl.when(s < NSUB - 1)
    def _():
        pltpu.sync_copy(hp_hbm.at[pl.ds(s * 312, 312)],
                        hp_sh.at[pl.ds(s * 312, 312)])

    @pl.when(s == NSUB - 1)
    def _():
        pltpu.sync_copy(hp_hbm.at[pl.ds(15 * 312, 320)],
                        hp_sh.at[pl.ds(15 * 312, 320)])

    zv = jnp.zeros((16,), jnp.float32)

    def zrow(i, carry):
        for jj in range(GROUPS):
            ms0[i, pl.ds(jj * 16, 16)] = zv
        return carry

    lax.fori_loop(0, CHUNK, zrow, 0)
    for kk in range(3):
        pltpu.sync_copy(ms0, table.at[pl.ds(s * 312 + kk * 80, 80)])

    @pl.when(s < NSUB - 1)
    def _():
        pltpu.sync_copy(ms0.at[pl.ds(0, 72)],
                        table.at[pl.ds(s * 312 + 240, 72)])

    @pl.when(s == NSUB - 1)
    def _():
        pltpu.sync_copy(ms0, table.at[pl.ds(15 * 312 + 240, 80)])

    plsc.subcore_barrier()

    def idx_load(g, q):
        pltpu.async_copy(src_hbm.at[s, g], si[q], isem[q])
        pltpu.async_copy(dst_hbm.at[c, s, g], di[q], isem[q])
        pltpu.async_copy(ew_hbm.at[c, s, g], wi[q], isem[q])
        pltpu.async_copy(pb_hbm.at[s, g], pb[q], isem[q])

    def idx_wait(q):
        pltpu.make_async_copy(src_hbm.at[s, 0], si[q], isem[q]).wait()
        pltpu.make_async_copy(dst_hbm.at[c, s, 0], di[q], isem[q]).wait()
        pltpu.make_async_copy(ew_hbm.at[c, s, 0], wi[q], isem[q]).wait()
        pltpu.make_async_copy(pb_hbm.at[s, 0], pb[q], isem[q]).wait()

    def gather_issue(q, i, p):
        pltpu.async_copy(hp_sh.at[si[q].at[i]], gb[p], gsem[p])

    def gather_wait(q, i, p):
        pltpu.make_async_copy(hp_sh.at[si[q].at[i]], gb[p], gsem[p]).wait()

    def scatter_issue(q, i, p):
        pltpu.async_copy(ms[p], table.at[di[q].at[i]], ssem[p], add=True)

    def scatter_wait(q, i, p):
        pltpu.make_async_copy(ms[p], table.at[di[q].at[i]], ssem[p]).wait()

    def weight(q, i, p):
        mask = jnp.int32(-65536)

        def wspill(e, carry):
            wv = wi[q][i, pl.ds(e * 16, 16)]
            bv = pb[q][i, pl.ds(e * 16, 16)]
            for lane in range(16):
                wsm[e * 16 + lane] = wv[lane]
                bsm[e * 16 + lane] = bv[lane]
            return carry

        lax.fori_loop(0, CHUNK // 16, wspill, 0)

        def edges(e4, carry):
            for d in range(4):
                edge = e4 * 4 + d
                w = wsm[edge]
                base = bsm[edge]
                for u in range(4):
                    vi = gb[p][edge, pl.ds(base + u * 16, 16)]
                    lo = lax.bitcast_convert_type(vi << 16, jnp.float32) * w
                    hi = lax.bitcast_convert_type(vi & mask, jnp.float32) * w
                    ms[p][edge, pl.ds(u * 32, 16)] = lo
                    ms[p][edge, pl.ds(u * 32 + 16, 16)] = hi
            return carry

        lax.fori_loop(0, CHUNK // 4, edges, 0)

    def group(g, carry):
        idx_load(g, 0)
        idx_wait(0)
        for i in range(GCH):
            pltpu.async_copy(hp_sh.at[si[0].at[i]], gb[0], gsem[0]).wait()
            weight(0, i, 0)
            pltpu.sync_copy(ms[0], table.at[di[0].at[i]], add=True)
        return carry

    lax.fori_loop(0, NG, group, 0)
    plsc.subcore_barrier()

    @pl.when(s < NSUB - 1)
    def _():
        pltpu.sync_copy(table.at[pl.ds(s * 312, 312)],
                        out_hbm.at[pl.ds(c * HALF + s * 312, 312)])

    @pl.when(s == NSUB - 1)
    def _():
        pltpu.sync_copy(table.at[pl.ds(15 * 312, 320)],
                        out_hbm.at[pl.ds(c * HALF + 15 * 312, 320)])


@functools.lru_cache(maxsize=1)
def _build_edge_pass():
    return functools.partial(
        pl.kernel,
        out_type=jax.ShapeDtypeStruct((N, C), jnp.float32),
        mesh=plsc.VectorSubcoreMesh(core_axis_name="c", subcore_axis_name="s"),
        scratch_types=[
            pltpu.VMEM((GCH, CHUNK), jnp.int32),
            pltpu.VMEM((GCH, CHUNK), jnp.int32),
            pltpu.VMEM((GCH, CHUNK), jnp.int32),
            pltpu.VMEM((GCH, CHUNK), jnp.int32),
            pltpu.VMEM((GCH, CHUNK), jnp.float32),
            pltpu.VMEM((GCH, CHUNK), jnp.float32),
            pltpu.VMEM((GCH, CHUNK), jnp.int32),
            pltpu.VMEM((GCH, CHUNK), jnp.int32),
            pltpu.VMEM((CHUNK, C), jnp.int32),
            pltpu.VMEM((CHUNK, C), jnp.int32),
            pltpu.VMEM((CHUNK, C), jnp.float32),
            pltpu.VMEM((CHUNK, C), jnp.float32),
            pltpu.VMEM_SHARED((HALF, C), jnp.float32),
            pltpu.VMEM_SHARED((N // 2, C), jnp.int32),
            pltpu.SMEM((CHUNK,), jnp.float32),
            pltpu.SMEM((CHUNK,), jnp.int32),
            pltpu.SemaphoreType.DMA,
            pltpu.SemaphoreType.DMA,
            pltpu.SemaphoreType.DMA,
            pltpu.SemaphoreType.DMA,
            pltpu.SemaphoreType.DMA,
            pltpu.SemaphoreType.DMA,
        ],
    )(_edge_body)


def _edge_pass(hp, src_p, dst_p, ew_p, pb_p):
    return _build_edge_pass()(hp, src_p, dst_p, ew_p, pb_p)


RBLK = 2000
GRID = N // RBLK

_MAVG = np.kron(np.eye(GROUPS, dtype=np.float32),
                np.ones((GSIZE, GSIZE), dtype=np.float32) / GSIZE)

_PERM = np.zeros((C, C), dtype=np.float32)
for _u in range(4):
    for _j in range(16):
        _PERM[32 * _u + _j, 32 * _u + 2 * _j] = 1.0
        _PERM[32 * _u + 16 + _j, 32 * _u + 2 * _j + 1] = 1.0


def _leaky(x):
    return jnp.where(x >= 0, x, 0.01 * x)


def _make_dense_call(n_in, use_gn, use_mm, want_sum, perm_first=False):
    def body(*refs):
        ins = refs[:n_in]
        k = n_in
        bias = refs[k][...]
        k += 1
        if perm_first:
            pm = refs[k][...]
            k += 1
        if use_gn:
            gamma = refs[k][...]; beta = refs[k + 1][...]; mavg = refs[k + 2][...]
            k += 3
        if use_mm:
            w = refs[k][...]
            k += 1
        outs = refs[k:]
        if perm_first:
            s = jnp.dot(ins[0][...], pm, preferred_element_type=jnp.float32)
        else:
            s = ins[0][...]
        for r in ins[1:]:
            s = s + r[...]
        s = s + bias
        if want_sum:
            outs[-1][...] = s
        if use_gn:
            m = jnp.dot(s, mavg, preferred_element_type=jnp.float32)
            xc = s - m
            var = jnp.dot(xc * xc, mavg, preferred_element_type=jnp.float32)
            y = xc * lax.rsqrt(var + EPS) * gamma + beta
            y = _leaky(y)
        else:
            y = s
        if use_mm:
            outs[0][...] = jnp.dot(
                y, w, preferred_element_type=jnp.float32).astype(jnp.bfloat16)
        elif not want_sum:
            outs[0][...] = y

    def call(inputs, bias, gn=None, w=None):
        in_specs = [pl.BlockSpec((RBLK, C), lambda i, o=off: (i + o, 0))
                    for (_, off) in inputs]
        args = [a for (a, _) in inputs]
        args.append(bias.reshape(1, -1))
        in_specs.append(pl.BlockSpec((1, C), lambda i: (0, 0)))
        if perm_first:
            args.append(jnp.asarray(_PERM))
            in_specs.append(pl.BlockSpec((C, C), lambda i: (0, 0)))
        if use_gn:
            gamma, beta = gn
            args += [gamma.reshape(1, -1), beta.reshape(1, -1),
                     jnp.asarray(_MAVG)]
            in_specs += [pl.BlockSpec((1, C), lambda i: (0, 0)),
                         pl.BlockSpec((1, C), lambda i: (0, 0)),
                         pl.BlockSpec((C, C), lambda i: (0, 0))]
        if use_mm:
            args.append(w)
            in_specs.append(pl.BlockSpec((C, C), lambda i: (0, 0)))
        out_shapes = []
        out_specs = []
        if use_mm:
            out_shapes.append(jax.ShapeDtypeStruct((N, C), jnp.bfloat16))
            out_specs.append(pl.BlockSpec((RBLK, C), lambda i: (i, 0)))
        elif not want_sum:
            out_shapes.append(jax.ShapeDtypeStruct((N, C), jnp.float32))
            out_specs.append(pl.BlockSpec((RBLK, C), lambda i: (i, 0)))
        if want_sum:
            out_shapes.append(jax.ShapeDtypeStruct((N, C), jnp.float32))
            out_specs.append(pl.BlockSpec((RBLK, C), lambda i: (i, 0)))
        return pl.pallas_call(
            body,
            grid=(GRID,),
            in_specs=in_specs,
            out_specs=out_specs if len(out_specs) > 1 else out_specs[0],
            out_shape=tuple(out_shapes) if len(out_shapes) > 1 else out_shapes[0],
        )(*args)

    return call


def _t_embed_body(t_ref, w_ref, b_ref, o_ref):
    lt = _leaky(t_ref[...])
    o_ref[...] = jnp.dot(lt, w_ref[...],
                         preferred_element_type=jnp.float32) + b_ref[...]


def _t_embed(t, wcat, bcat):
    return pl.pallas_call(
        _t_embed_body,
        grid=(GRID,),
        in_specs=[pl.BlockSpec((RBLK, C), lambda i: (i, 0)),
                  pl.BlockSpec((C, 4 * C), lambda i: (0, 0)),
                  pl.BlockSpec((1, 4 * C), lambda i: (0, 0))],
        out_specs=pl.BlockSpec((RBLK, 4 * C), lambda i: (i, 0)),
        out_shape=jax.ShapeDtypeStruct((N, 4 * C), jnp.float32),
    )(t, wcat, bcat.reshape(1, -1))


def _pack_bf16(h16):
    return lax.bitcast_convert_type(h16.reshape(N // 2, C, 2), jnp.int32)


def kernel(x, t, edge_index, edge_weight, params):
    src = edge_index[0].astype(jnp.int32)
    dst = edge_index[1].astype(jnp.int32)
    ew = edge_weight.astype(jnp.float32)
    pad = E_PAD - E
    src_f = jnp.concatenate([src, jnp.zeros((pad,), jnp.int32)])
    dst_f = jnp.concatenate([dst, jnp.zeros((pad,), jnp.int32)])
    ew_f = jnp.concatenate([ew, jnp.zeros((pad,), jnp.float32)])
    spread = src_f >> 1
    in0 = dst_f < HALF
    dst0 = jnp.where(in0, dst_f, spread)
    ew0 = jnp.where(in0, ew_f, 0.0)
    dst1 = jnp.where(in0, spread, dst_f - HALF)
    ew1 = jnp.where(in0, 0.0, ew_f)
    src_p = (src_f >> 1).reshape(NSUB, NG, GCH, CHUNK)
    pb_p = ((src_f & 1) << 6).reshape(NSUB, NG, GCH, CHUNK)
    dst_p = jnp.stack([dst0, dst1]).reshape(2, NSUB, NG, GCH, CHUNK)
    ew_p = jnp.stack([ew0, ew1]).reshape(2, NSUB, NG, GCH, CHUNK)

    wtcat = jnp.concatenate([p['Wt'] for p in params], axis=1)
    btcat = jnp.concatenate([p['bt'] for p in params])
    tts = _t_embed(t, wtcat, btcat)
    tt = [lax.slice(tts, (0, b * C), (N, (b + 1) * C)) for b in range(4)]

    gn_mm_1 = _make_dense_call(1, True, True, False)
    gn_mm_2p = _make_dense_call(2, True, True, False, perm_first=True)
    gn_mm_2ps = _make_dense_call(2, True, True, True, perm_first=True)
    gn_mm_3ps = _make_dense_call(3, True, True, True, perm_first=True)
    sum_2p = _make_dense_call(2, False, False, False, perm_first=True)

    def econv(h16):
        return _edge_pass(_pack_bf16(h16), src_p, dst_p, ew_p, pb_p)

    zb = jnp.zeros((C,), jnp.float32)
    p0, p1, p2, p3 = params

    u1 = gn_mm_1([(x, 0)], zb, gn=(p0['gn1_g'], p0['gn1_b']), w=p0['W1'])
    A1 = econv(u1)
    v1 = gn_mm_2p([(A1, 0), (tt[0], 0)], p0['b1'],
                  gn=(p0['gn2_g'], p0['gn2_b']), w=p0['W2'])
    Q1 = econv(v1)

    u2, h1 = gn_mm_2ps([(Q1, 0), (x, 0)], p0['b2'],
                       gn=(p1['gn1_g'], p1['gn1_b']), w=p1['W1'])
    P2 = econv(u2)
    v2 = gn_mm_2p([(P2, 0), (tt[1], 0)], p1['b1'],
                  gn=(p1['gn2_g'], p1['gn2_b']), w=p1['W2'])
    Q2 = econv(v2)

    u3, h2 = gn_mm_2ps([(Q2, 0), (h1, 0)], p1['b2'],
                       gn=(p2['gn1_g'], p2['gn1_b']), w=p2['W1'])
    P3 = econv(u3)
    v3 = gn_mm_2p([(P3, 0), (tt[2], 0)], p2['b1'],
                  gn=(p2['gn2_g'], p2['gn2_b']), w=p2['W2'])
    Q3 = econv(v3)

    u4, s4 = gn_mm_3ps([(Q3, 0), (h2, 0), (h1, 0)], p2['b2'],
                       gn=(p3['gn1_g'], p3['gn1_b']), w=p3['W1'])
    P4 = econv(u4)
    v4 = gn_mm_2p([(P4, 0), (tt[3], 0)], p3['b1'],
                  gn=(p3['gn2_g'], p3['gn2_b']), w=p3['W2'])
    Q4 = econv(v4)

    return sum_2p([(Q4, 0), (s4, 0)], p3['b2'])


# speedup vs baseline: 1.3113x; 1.3113x over previous
import functools

import numpy as np
import jax
import jax.numpy as jnp
from jax import lax
from jax.experimental import pallas as pl
from jax.experimental.pallas import tpu as pltpu
from jax.experimental.pallas import tpu_sc as plsc

N = 10000
C = 128
E = 320000
HALF = N // 2
GROUPS = 8
GSIZE = C // GROUPS
EPS = 1e-5

NCORES = 2
NSUB = 16
CHUNK = 80
GCH = 4
NG = 64
NCH = NG * GCH
EPT = NCH * CHUNK
E_PAD = EPT * NSUB
NPAIR = NG // 2


def _edge_body(hp_hbm, src_hbm, dst_hbm, ew_hbm, pb_hbm, out_hbm,
               si0, si1, di0, di1, wi0, wi1, pb0, pb1,
               gb0, gb1, ms0, ms1, table, hp_sh, wsm, bsm,
               isem0, isem1, gsem0, gsem1, ssem0, ssem1):
    c = lax.axis_index("c")
    s = lax.axis_index("s")
    si = [si0, si1]
    di = [di0, di1]
    wi = [wi0, wi1]
    pb = [pb0, pb1]
    gb = [gb0, gb1]
    ms = [ms0, ms1]
    isem = [isem0, isem1]
    gsem = [gsem0, gsem1]
    ssem = [ssem0, ssem1]

    @pl.when(s < NSUB - 1)
    def _():
        pltpu.sync_copy(hp_hbm.at[pl.ds(s * 312, 312)],
                        hp_sh.at[pl.ds(s * 312, 312)])

    @pl.when(s == NSUB - 1)
    def _():
        pltpu.sync_copy(hp_hbm.at[pl.ds(15 * 312, 320)],
                        hp_sh.at[pl.ds(15 * 312, 320)])

    zv = jnp.zeros((16,), jnp.float32)

    def zrow(i, carry):
        for jj in range(GROUPS):
            ms0[i, pl.ds(jj * 16, 16)] = zv
        return carry

    lax.fori_loop(0, CHUNK, zrow, 0)
    for kk in range(3):
        pltpu.sync_copy(ms0, table.at[pl.ds(s * 312 + kk * 80, 80)])

    @pl.when(s < NSUB - 1)
    def _():
        pltpu.sync_copy(ms0.at[pl.ds(0, 72)],
                        table.at[pl.ds(s * 312 + 240, 72)])

    @pl.when(s == NSUB - 1)
    def _():
        pltpu.sync_copy(ms0, table.at[pl.ds(15 * 312 + 240, 80)])

    plsc.subcore_barrier()

    def idx_load(g, q):
        pltpu.async_copy(src_hbm.at[s, g], si[q], isem[q])
        pltpu.async_copy(dst_hbm.at[c, s, g], di[q], isem[q])
        pltpu.async_copy(ew_hbm.at[c, s, g], wi[q], isem[q])
        pltpu.async_copy(pb_hbm.at[s, g], pb[q], isem[q])

    def idx_wait(q):
        pltpu.make_async_copy(src_hbm.at[s, 0], si[q], isem[q]).wait()
        pltpu.make_async_copy(dst_hbm.at[c, s, 0], di[q], isem[q]).wait()
        pltpu.make_async_copy(ew_hbm.at[c, s, 0], wi[q], isem[q]).wait()
        pltpu.make_async_copy(pb_hbm.at[s, 0], pb[q], isem[q]).wait()

    def gather_issue(q, i, p):
        pltpu.async_copy(hp_sh.at[si[q].at[i]], gb[p], gsem[p])

    def gather_wait(q, i, p):
        pltpu.make_async_copy(hp_sh.at[si[q].at[i]], gb[p], gsem[p]).wait()

    def scatter_issue(q, i, p):
        pltpu.async_copy(ms[p], table.at[di[q].at[i]], ssem[p], add=True)

    def scatter_wait(q, i, p):
        pltpu.make_async_copy(ms[p], table.at[di[q].at[i]], ssem[p]).wait()

    def weight(q, i, p):
        mask = jnp.int32(-65536)

        def wspill(e, carry):
            wv = wi[q][i, pl.ds(e * 16, 16)]
            bv = pb[q][i, pl.ds(e * 16, 16)]
            for lane in range(16):
                wsm[e * 16 + lane] = wv[lane]
                bsm[e * 16 + lane] = bv[lane]
            return carry

        lax.fori_loop(0, CHUNK // 16, wspill, 0)

        def edges(e4, carry):
            for d in range(4):
                edge = e4 * 4 + d
                w = wsm[edge]
                base = bsm[edge]
                for u in range(4):
                    vi = gb[p][edge, pl.ds(base + u * 16, 16)]
                    lo = lax.bitcast_convert_type(vi << 16, jnp.float32) * w
                    hi = lax.bitcast_convert_type(vi & mask, jnp.float32) * w
                    ms[p][edge, pl.ds(u * 32, 16)] = lo
                    ms[p][edge, pl.ds(u * 32 + 16, 16)] = hi
            return carry

        lax.fori_loop(0, CHUNK // 4, edges, 0)

    idx_load(0, 0)
    idx_wait(0)
    gather_issue(0, 0, 0)
    gather_issue(0, 1, 1)

    def pair(m, carry):
        for t in range(8):
            gg = t // 4
            i = t % 4
            p = t % 2
            q = gg
            if t >= 2:
                i2 = (t - 2) % 4
                q2 = (t - 2) // 4
                scatter_wait(q2, i2, p)
            else:
                @pl.when(m >= 1)
                def _(i2=(t + 6) % 4, p2=p):
                    scatter_wait(1, i2, p2)
            if i == 1:
                if gg == 0:
                    idx_load(2 * m + 1, 1)
                else:
                    @pl.when(m < NPAIR - 1)
                    def _(mm=m):
                        idx_load(2 * mm + 2, 0)
            gather_wait(q, i, p)
            weight(q, i, p)
            if i == 2:
                if gg == 0:
                    idx_wait(1)
                else:
                    @pl.when(m < NPAIR - 1)
                    def _():
                        idx_wait(0)
            scatter_issue(q, i, p)
            kp2_i = (t + 2) % 4
            if t < 6:
                gather_issue((t + 2) // 4, kp2_i, p)
            else:
                @pl.when(m < NPAIR - 1)
                def _(ii=kp2_i, pp=p):
                    gather_issue(0, ii, pp)
        return carry

    lax.fori_loop(0, NPAIR, pair, 0)

    scatter_wait(1, 2, 0)
    scatter_wait(1, 3, 1)
    plsc.subcore_barrier()

    @pl.when(s < NSUB - 1)
    def _():
        pltpu.sync_copy(table.at[pl.ds(s * 312, 312)],
                        out_hbm.at[pl.ds(c * HALF + s * 312, 312)])

    @pl.when(s == NSUB - 1)
    def _():
        pltpu.sync_copy(table.at[pl.ds(15 * 312, 320)],
                        out_hbm.at[pl.ds(c * HALF + 15 * 312, 320)])


@functools.lru_cache(maxsize=1)
def _build_edge_pass():
    return functools.partial(
        pl.kernel,
        out_type=jax.ShapeDtypeStruct((N, C), jnp.float32),
        mesh=plsc.VectorSubcoreMesh(core_axis_name="c", subcore_axis_name="s"),
        scratch_types=[
            pltpu.VMEM((GCH, CHUNK), jnp.int32),
            pltpu.VMEM((GCH, CHUNK), jnp.int32),
            pltpu.VMEM((GCH, CHUNK), jnp.int32),
            pltpu.VMEM((GCH, CHUNK), jnp.int32),
            pltpu.VMEM((GCH, CHUNK), jnp.float32),
            pltpu.VMEM((GCH, CHUNK), jnp.float32),
            pltpu.VMEM((GCH, CHUNK), jnp.int32),
            pltpu.VMEM((GCH, CHUNK), jnp.int32),
            pltpu.VMEM((CHUNK, C), jnp.int32),
            pltpu.VMEM((CHUNK, C), jnp.int32),
            pltpu.VMEM((CHUNK, C), jnp.float32),
            pltpu.VMEM((CHUNK, C), jnp.float32),
            pltpu.VMEM_SHARED((HALF, C), jnp.float32),
            pltpu.VMEM_SHARED((N // 2, C), jnp.int32),
            pltpu.SMEM((CHUNK,), jnp.float32),
            pltpu.SMEM((CHUNK,), jnp.int32),
            pltpu.SemaphoreType.DMA,
            pltpu.SemaphoreType.DMA,
            pltpu.SemaphoreType.DMA,
            pltpu.SemaphoreType.DMA,
            pltpu.SemaphoreType.DMA,
            pltpu.SemaphoreType.DMA,
        ],
    )(_edge_body)


def _edge_pass(hp, src_p, dst_p, ew_p, pb_p):
    return _build_edge_pass()(hp, src_p, dst_p, ew_p, pb_p)


RBLK = 2000
GRID = N // RBLK

_MAVG = np.kron(np.eye(GROUPS, dtype=np.float32),
                np.ones((GSIZE, GSIZE), dtype=np.float32) / GSIZE)

_PERM = np.zeros((C, C), dtype=np.float32)
for _u in range(4):
    for _j in range(16):
        _PERM[32 * _u + _j, 32 * _u + 2 * _j] = 1.0
        _PERM[32 * _u + 16 + _j, 32 * _u + 2 * _j + 1] = 1.0


def _leaky(x):
    return jnp.where(x >= 0, x, 0.01 * x)


def _make_dense_call(n_in, use_gn, use_mm, want_sum, perm_first=False):
    def body(*refs):
        ins = refs[:n_in]
        k = n_in
        bias = refs[k][...]
        k += 1
        if perm_first:
            pm = refs[k][...]
            k += 1
        if use_gn:
            gamma = refs[k][...]; beta = refs[k + 1][...]; mavg = refs[k + 2][...]
            k += 3
        if use_mm:
            w = refs[k][...]
            k += 1
        outs = refs[k:]
        if perm_first:
            s = jnp.dot(ins[0][...], pm, preferred_element_type=jnp.float32)
        else:
            s = ins[0][...]
        for r in ins[1:]:
            s = s + r[...]
        s = s + bias
        if want_sum:
            outs[-1][...] = s
        if use_gn:
            m = jnp.dot(s, mavg, preferred_element_type=jnp.float32)
            xc = s - m
            var = jnp.dot(xc * xc, mavg, preferred_element_type=jnp.float32)
            y = xc * lax.rsqrt(var + EPS) * gamma + beta
            y = _leaky(y)
        else:
            y = s
        if use_mm:
            outs[0][...] = jnp.dot(
                y, w, preferred_element_type=jnp.float32).astype(jnp.bfloat16)
        elif not want_sum:
            outs[0][...] = y

    def call(inputs, bias, gn=None, w=None):
        in_specs = [pl.BlockSpec((RBLK, C), lambda i, o=off: (i + o, 0))
                    for (_, off) in inputs]
        args = [a for (a, _) in inputs]
        args.append(bias.reshape(1, -1))
        in_specs.append(pl.BlockSpec((1, C), lambda i: (0, 0)))
        if perm_first:
            args.append(jnp.asarray(_PERM))
            in_specs.append(pl.BlockSpec((C, C), lambda i: (0, 0)))
        if use_gn:
            gamma, beta = gn
            args += [gamma.reshape(1, -1), beta.reshape(1, -1),
                     jnp.asarray(_MAVG)]
            in_specs += [pl.BlockSpec((1, C), lambda i: (0, 0)),
                         pl.BlockSpec((1, C), lambda i: (0, 0)),
                         pl.BlockSpec((C, C), lambda i: (0, 0))]
        if use_mm:
            args.append(w)
            in_specs.append(pl.BlockSpec((C, C), lambda i: (0, 0)))
        out_shapes = []
        out_specs = []
        if use_mm:
            out_shapes.append(jax.ShapeDtypeStruct((N, C), jnp.bfloat16))
            out_specs.append(pl.BlockSpec((RBLK, C), lambda i: (i, 0)))
        elif not want_sum:
            out_shapes.append(jax.ShapeDtypeStruct((N, C), jnp.float32))
            out_specs.append(pl.BlockSpec((RBLK, C), lambda i: (i, 0)))
        if want_sum:
            out_shapes.append(jax.ShapeDtypeStruct((N, C), jnp.float32))
            out_specs.append(pl.BlockSpec((RBLK, C), lambda i: (i, 0)))
        return pl.pallas_call(
            body,
            grid=(GRID,),
            in_specs=in_specs,
            out_specs=out_specs if len(out_specs) > 1 else out_specs[0],
            out_shape=tuple(out_shapes) if len(out_shapes) > 1 else out_shapes[0],
        )(*args)

    return call


def _t_embed_body(t_ref, w_ref, b_ref, o_ref):
    lt = _leaky(t_ref[...])
    o_ref[...] = jnp.dot(lt, w_ref[...],
                         preferred_element_type=jnp.float32) + b_ref[...]


def _t_embed(t, wcat, bcat):
    return pl.pallas_call(
        _t_embed_body,
        grid=(GRID,),
        in_specs=[pl.BlockSpec((RBLK, C), lambda i: (i, 0)),
                  pl.BlockSpec((C, 4 * C), lambda i: (0, 0)),
                  pl.BlockSpec((1, 4 * C), lambda i: (0, 0))],
        out_specs=pl.BlockSpec((RBLK, 4 * C), lambda i: (i, 0)),
        out_shape=jax.ShapeDtypeStruct((N, 4 * C), jnp.float32),
    )(t, wcat, bcat.reshape(1, -1))


def _pack_bf16(h16):
    return lax.bitcast_convert_type(h16.reshape(N // 2, C, 2), jnp.int32)


def kernel(x, t, edge_index, edge_weight, params):
    src = edge_index[0].astype(jnp.int32)
    dst = edge_index[1].astype(jnp.int32)
    ew = edge_weight.astype(jnp.float32)
    pad = E_PAD - E
    src_f = jnp.concatenate([src, jnp.zeros((pad,), jnp.int32)])
    dst_f = jnp.concatenate([dst, jnp.zeros((pad,), jnp.int32)])
    ew_f = jnp.concatenate([ew, jnp.zeros((pad,), jnp.float32)])
    spread = src_f >> 1
    in0 = dst_f < HALF
    dst0 = jnp.where(in0, dst_f, spread)
    ew0 = jnp.where(in0, ew_f, 0.0)
    dst1 = jnp.where(in0, spread, dst_f - HALF)
    ew1 = jnp.where(in0, 0.0, ew_f)
    src_p = (src_f >> 1).reshape(NSUB, NG, GCH, CHUNK)
    pb_p = ((src_f & 1) << 6).reshape(NSUB, NG, GCH, CHUNK)
    dst_p = jnp.stack([dst0, dst1]).reshape(2, NSUB, NG, GCH, CHUNK)
    ew_p = jnp.stack([ew0, ew1]).reshape(2, NSUB, NG, GCH, CHUNK)

    wtcat = jnp.concatenate([p['Wt'] for p in params], axis=1)
    btcat = jnp.concatenate([p['bt'] for p in params])
    tts = _t_embed(t, wtcat, btcat)
    tt = [lax.slice(tts, (0, b * C), (N, (b + 1) * C)) for b in range(4)]

    gn_mm_1 = _make_dense_call(1, True, True, False)
    gn_mm_2p = _make_dense_call(2, True, True, False, perm_first=True)
    gn_mm_2ps = _make_dense_call(2, True, True, True, perm_first=True)
    gn_mm_3ps = _make_dense_call(3, True, True, True, perm_first=True)
    sum_2p = _make_dense_call(2, False, False, False, perm_first=True)

    def econv(h16):
        return _edge_pass(_pack_bf16(h16), src_p, dst_p, ew_p, pb_p)

    zb = jnp.zeros((C,), jnp.float32)
    p0, p1, p2, p3 = params

    u1 = gn_mm_1([(x, 0)], zb, gn=(p0['gn1_g'], p0['gn1_b']), w=p0['W1'])
    A1 = econv(u1)
    v1 = gn_mm_2p([(A1, 0), (tt[0], 0)], p0['b1'],
                  gn=(p0['gn2_g'], p0['gn2_b']), w=p0['W2'])
    Q1 = econv(v1)

    u2, h1 = gn_mm_2ps([(Q1, 0), (x, 0)], p0['b2'],
                       gn=(p1['gn1_g'], p1['gn1_b']), w=p1['W1'])
    P2 = econv(u2)
    v2 = gn_mm_2p([(P2, 0), (tt[1], 0)], p1['b1'],
                  gn=(p1['gn2_g'], p1['gn2_b']), w=p1['W2'])
    Q2 = econv(v2)

    u3, h2 = gn_mm_2ps([(Q2, 0), (h1, 0)], p1['b2'],
                       gn=(p2['gn1_g'], p2['gn1_b']), w=p2['W1'])
    P3 = econv(u3)
    v3 = gn_mm_2p([(P3, 0), (tt[2], 0)], p2['b1'],
                  gn=(p2['gn2_g'], p2['gn2_b']), w=p2['W2'])
    Q3 = econv(v3)

    u4, s4 = gn_mm_3ps([(Q3, 0), (h2, 0), (h1, 0)], p2['b2'],
                       gn=(p3['gn1_g'], p3['gn1_b']), w=p3['W1'])
    P4 = econv(u4)
    v4 = gn_mm_2p([(P4, 0), (tt[3], 0)], p3['b1'],
                  gn=(p3['gn2_g'], p3['gn2_b']), w=p3['W2'])
    Q4 = econv(v4)

    return sum_2p([(Q4, 0), (s4, 0)], p3['b2'])
